# baseline (speedup 1.0000x reference)
import functools

import jax
import jax.numpy as jnp
from jax import lax
from jax.experimental import pallas as pl
from jax.experimental.pallas import tpu as pltpu
from jax.experimental.pallas import tpu_sc as plsc

_B = 4
_L = 4096
_D = 1024

_NUM_CORES = 2
_NUM_SUBCORES = 16
_NW = _NUM_CORES * _NUM_SUBCORES
_ROWS_PER_W = _L // _NW
_CHUNKS = ((0, 8), (8, 112), (120, 8))


def _bcast_body(table_hbm, out_hbm, buf0, buf1,
                isem0, isem1, osem0, osem1):
    wid = lax.axis_index("s") * _NUM_CORES + lax.axis_index("c")
    base = wid * _ROWS_PER_W
    bufs = (buf0, buf1, buf0)
    isems = (isem0, isem1, isem0)
    osems = (osem0, osem1, osem0)

    def gather_in(i):
        off, rows = _CHUNKS[i]
        return pltpu.async_copy(
            table_hbm.at[pl.ds(base + off, rows), :], bufs[i], isems[i])

    def scatter_out(i):
        off, rows = _CHUNKS[i]
        return [
            pltpu.async_copy(
                bufs[i], out_hbm.at[pl.ds(b * _L + base + off, rows), :],
                osems[i])
            for b in range(_B)
        ]

    in0 = gather_in(0)
    in1 = gather_in(1)
    in0.wait()
    s0 = scatter_out(0)
    in1.wait()
    s1 = scatter_out(1)
    for c in s0:
        c.wait()
    in2 = gather_in(2)
    in2.wait()
    s2 = scatter_out(2)
    for c in s1 + s2:
        c.wait()


_bcast = functools.partial(
    pl.kernel,
    mesh=plsc.VectorSubcoreMesh(core_axis_name="c", subcore_axis_name="s"),
    out_type=jax.ShapeDtypeStruct((_B * _L, _D), jnp.float32),
    scratch_types=[
        pltpu.VMEM((_CHUNKS[0][1], _D), jnp.float32),
        pltpu.VMEM((_CHUNKS[1][1], _D), jnp.float32),
        pltpu.SemaphoreType.DMA,
        pltpu.SemaphoreType.DMA,
        pltpu.SemaphoreType.DMA,
        pltpu.SemaphoreType.DMA,
    ],
)(_bcast_body)


def kernel(words_embedding, pos_table):
    del words_embedding
    out = _bcast(pos_table)
    return out.reshape(_B, _L, _D)


# speedup vs baseline: 1.0007x; 1.0007x over previous
import functools

import jax
import jax.numpy as jnp
from jax import lax
from jax.experimental import pallas as pl
from jax.experimental.pallas import tpu as pltpu
from jax.experimental.pallas import tpu_sc as plsc

_B = 4
_L = 4096
_D = 1024

_NUM_CORES = 2
_NUM_SUBCORES = 16
_NW = _NUM_CORES * _NUM_SUBCORES
_ROWS_PER_W = _L // _NW
_CHUNKS = ((0, 16), (16, 96), (112, 16))


def _bcast_body(table_hbm, out_hbm, buf0, buf1,
                isem0, isem1, osem0, osem1):
    wid = lax.axis_index("s") * _NUM_CORES + lax.axis_index("c")
    base = wid * _ROWS_PER_W
    bufs = (buf0, buf1, buf0)
    isems = (isem0, isem1, isem0)
    osems = (osem0, osem1, osem0)

    def gather_in(i):
        off, rows = _CHUNKS[i]
        return pltpu.async_copy(
            table_hbm.at[pl.ds(base + off, rows), :], bufs[i], isems[i])

    def scatter_out(i):
        off, rows = _CHUNKS[i]
        return [
            pltpu.async_copy(
                bufs[i], out_hbm.at[pl.ds(b * _L + base + off, rows), :],
                osems[i])
            for b in range(_B)
        ]

    in0 = gather_in(0)
    in1 = gather_in(1)
    in0.wait()
    s0 = scatter_out(0)
    in1.wait()
    s1 = scatter_out(1)
    for c in s0:
        c.wait()
    in2 = gather_in(2)
    in2.wait()
    s2 = scatter_out(2)
    for c in s1 + s2:
        c.wait()


_bcast = functools.partial(
    pl.kernel,
    mesh=plsc.VectorSubcoreMesh(core_axis_name="c", subcore_axis_name="s"),
    out_type=jax.ShapeDtypeStruct((_B * _L, _D), jnp.float32),
    scratch_types=[
        pltpu.VMEM((_CHUNKS[0][1], _D), jnp.float32),
        pltpu.VMEM((_CHUNKS[1][1], _D), jnp.float32),
        pltpu.SemaphoreType.DMA,
        pltpu.SemaphoreType.DMA,
        pltpu.SemaphoreType.DMA,
        pltpu.SemaphoreType.DMA,
    ],
)(_bcast_body)


def kernel(words_embedding, pos_table):
    del words_embedding
    out = _bcast(pos_table)
    return out.reshape(_B, _L, _D)


# speedup vs baseline: 1.0009x; 1.0003x over previous
import functools

import jax
import jax.numpy as jnp
from jax import lax
from jax.experimental import pallas as pl
from jax.experimental.pallas import tpu as pltpu
from jax.experimental.pallas import tpu_sc as plsc

_B = 4
_L = 4096
_D = 1024

_NUM_CORES = 2
_NUM_SUBCORES = 16
_NW = _NUM_CORES * _NUM_SUBCORES
_ROWS_PER_W = _L // _NW
_CHUNKS = ((0, 8), (8, 112), (120, 8))


def _bcast_body(table_hbm, out_hbm, buf0, buf1,
                isem0, isem1, osem0, osem1):
    wid = lax.axis_index("s") * _NUM_CORES + lax.axis_index("c")
    base = wid * _ROWS_PER_W
    bufs = (buf0, buf1, buf0)
    isems = (isem0, isem1, isem0)
    osems = (osem0, osem1, osem0)

    def gather_in(i):
        off, rows = _CHUNKS[i]
        return pltpu.async_copy(
            table_hbm.at[pl.ds(base + off, rows), :], bufs[i], isems[i])

    def scatter_out(i):
        off, rows = _CHUNKS[i]
        return [
            pltpu.async_copy(
                bufs[i], out_hbm.at[pl.ds(b * _L + base + off, rows), :],
                osems[i])
            for b in range(_B)
        ]

    in0 = gather_in(0)
    in1 = gather_in(1)
    in0.wait()
    s0 = scatter_out(0)
    in1.wait()
    s1 = scatter_out(1)
    for c in s0:
        c.wait()
    in2 = gather_in(2)
    in2.wait()
    s2 = scatter_out(2)
    for c in s1 + s2:
        c.wait()


_bcast = functools.partial(
    pl.kernel,
    mesh=plsc.VectorSubcoreMesh(core_axis_name="c", subcore_axis_name="s"),
    out_type=jax.ShapeDtypeStruct((_B * _L, _D), jnp.float32),
    scratch_types=[
        pltpu.VMEM((_CHUNKS[0][1], _D), jnp.float32),
        pltpu.VMEM((_CHUNKS[1][1], _D), jnp.float32),
        pltpu.SemaphoreType.DMA,
        pltpu.SemaphoreType.DMA,
        pltpu.SemaphoreType.DMA,
        pltpu.SemaphoreType.DMA,
    ],
)(_bcast_body)


def kernel(words_embedding, pos_table):
    del words_embedding
    out = _bcast(pos_table)
    return out.reshape(_B, _L, _D)
